# baseline (speedup 1.0000x reference)
import functools

import jax
import jax.numpy as jnp
from jax import lax
from jax.experimental import pallas as pl
from jax.experimental.pallas import tpu as pltpu
from jax.experimental.pallas import tpu_sc as plsc

NUM_TRIGGERS = 4
BATCH = 32
SEQ = 2048
NUM_STATES = 64

T = BATCH * SEQ
NC, NS, L = 2, 16, 16
NW = NC * NS
TOK_PER_W = T // NW
BT = 128
NBLK = TOK_PER_W // BT
BLK_ELEMS = BT * NUM_STATES


def _sc_body(scores_hbm, out_hbm, buf, outbuf, sem0, sem1):
    wid = lax.axis_index("s") * NC + lax.axis_index("c")
    tok0 = wid * TOK_PER_W
    iota = lax.iota(jnp.int32, L)
    sems = (sem0, sem1)

    def start(blk, slot):
        s0 = (tok0 + blk * BT) * NUM_STATES
        for t in range(NUM_TRIGGERS):
            pltpu.async_copy(
                scores_hbm.at[t, pl.ds(s0, BLK_ELEMS)], buf.at[slot, t], sems[slot]
            )

    def drain(slot):
        for t in range(NUM_TRIGGERS):
            pltpu.make_async_copy(
                scores_hbm.at[t, pl.ds(0, BLK_ELEMS)], buf.at[slot, t], sems[slot]
            ).wait()

    def compute(blk, slot):
        @pl.loop(0, BT // L)
        def _g(g):
            base = g * (L * NUM_STATES)
            acc = jnp.zeros((L,), jnp.int32)
            for k in range(L):
                tb = base + k * NUM_STATES
                cands = []
                for gg in range(NUM_STATES // L):
                    s = tb + gg * L
                    m = jnp.minimum(
                        jnp.minimum(
                            buf[slot, 0, pl.ds(s, L)], buf[slot, 1, pl.ds(s, L)]
                        ),
                        jnp.minimum(
                            buf[slot, 2, pl.ds(s, L)], buf[slot, 3, pl.ds(s, L)]
                        ),
                    )
                    f = plsc.all_reduce_ffs(m > 0.0)
                    cands.append(jnp.where(f >= L, NUM_STATES, f + gg * L))
                best = jnp.minimum(
                    jnp.minimum(cands[0], cands[1]),
                    jnp.minimum(cands[2], cands[3]),
                )
                best = jnp.where(best == NUM_STATES, 0, best)
                acc = jnp.where(iota == k, best, acc)
            outbuf[pl.ds(blk * BT + g * L, L)] = acc.astype(jnp.float32)

    start(0, 0)
    start(1, 1)

    @pl.loop(0, NBLK, step=2)
    def _blk(blk):
        for b in range(2):
            drain(b)
            compute(blk + b, b)

            @pl.when(blk + b + 2 < NBLK)
            def _():
                s0 = (tok0 + (blk + b + 2) * BT) * NUM_STATES
                for t in range(NUM_TRIGGERS):
                    pltpu.async_copy(
                        scores_hbm.at[t, pl.ds(s0, BLK_ELEMS)],
                        buf.at[b, t],
                        sems[b],
                    )

    pltpu.sync_copy(outbuf, out_hbm.at[pl.ds(tok0, TOK_PER_W)])


@jax.jit
def _run(scores_flat):
    mesh = plsc.VectorSubcoreMesh(
        core_axis_name="c", subcore_axis_name="s", num_cores=NC, num_subcores=NS
    )
    return pl.kernel(
        _sc_body,
        out_type=jax.ShapeDtypeStruct((T,), jnp.float32),
        mesh=mesh,
        compiler_params=pltpu.CompilerParams(
            needs_layout_passes=False, disable_bounds_checks=True
        ),
        scratch_types=[
            pltpu.VMEM((2, NUM_TRIGGERS, BLK_ELEMS), jnp.float32),
            pltpu.VMEM((TOK_PER_W,), jnp.float32),
            pltpu.SemaphoreType.DMA,
            pltpu.SemaphoreType.DMA,
        ],
    )(scores_flat)


TC_TB = 8192
TC_NB = T // TC_TB


def _tc_body(x_ref, o_ref, m_ref):
    t = pl.program_id(1)
    x = x_ref[0]

    @pl.when(t == 0)
    def _():
        m_ref[...] = x

    @pl.when(t > 0)
    def _():
        m_ref[...] = jnp.minimum(m_ref[...], x)

    @pl.when(t == NUM_TRIGGERS - 1)
    def _():
        m = m_ref[...]
        iota = lax.broadcasted_iota(
            jnp.int32, (TC_TB, NUM_STATES), 1
        ).astype(jnp.float32)
        cand = jnp.where(m > 0.0, iota, jnp.float32(NUM_STATES))
        idx = jnp.min(cand, axis=-1)
        idx = jnp.where(idx == jnp.float32(NUM_STATES), jnp.float32(0.0), idx)
        o_ref[...] = idx.reshape(1, 1, TC_TB)


@jax.jit
def _run_tc(scores3):
    return pl.pallas_call(
        _tc_body,
        grid=(TC_NB, NUM_TRIGGERS),
        in_specs=[
            pl.BlockSpec((1, TC_TB, NUM_STATES), lambda i, t: (t, i, 0))
        ],
        out_specs=pl.BlockSpec((1, 1, TC_TB), lambda i, t: (i, 0, 0)),
        out_shape=jax.ShapeDtypeStruct((TC_NB, 1, TC_TB), jnp.float32),
        scratch_shapes=[pltpu.VMEM((TC_TB, NUM_STATES), jnp.float32)],
        compiler_params=pltpu.CompilerParams(
            dimension_semantics=("arbitrary", "arbitrary")
        ),
    )(scores3)


def kernel(trigger_scores):
    scores3 = trigger_scores.reshape(NUM_TRIGGERS, T, NUM_STATES)
    return _run_tc(scores3).reshape(BATCH, SEQ)


# speedup vs baseline: 5.5146x; 5.5146x over previous
import functools

import jax
import jax.numpy as jnp
from jax import lax
from jax.experimental import pallas as pl
from jax.experimental.pallas import tpu as pltpu
from jax.experimental.pallas import tpu_sc as plsc

NUM_TRIGGERS = 4
BATCH = 32
SEQ = 2048
NUM_STATES = 64

T = BATCH * SEQ
NC, NS, L = 2, 16, 16
NW = NC * NS
TOK_PER_W = T // NW
BT = 128
NBLK = TOK_PER_W // BT
BLK_ELEMS = BT * NUM_STATES


def _sc_body(scores_hbm, out_hbm, buf, outbuf, sem0, sem1):
    wid = lax.axis_index("s") * NC + lax.axis_index("c")
    tok0 = wid * TOK_PER_W
    iota = lax.iota(jnp.int32, L)
    sems = (sem0, sem1)

    def start(blk, slot):
        s0 = (tok0 + blk * BT) * NUM_STATES
        for t in range(NUM_TRIGGERS):
            pltpu.async_copy(
                scores_hbm.at[t, pl.ds(s0, BLK_ELEMS)], buf.at[slot, t], sems[slot]
            )

    def drain(slot):
        for t in range(NUM_TRIGGERS):
            pltpu.make_async_copy(
                scores_hbm.at[t, pl.ds(0, BLK_ELEMS)], buf.at[slot, t], sems[slot]
            ).wait()

    def compute(blk, slot):
        @pl.loop(0, BT // L)
        def _g(g):
            base = g * (L * NUM_STATES)
            acc = jnp.zeros((L,), jnp.int32)
            for k in range(L):
                tb = base + k * NUM_STATES
                cands = []
                for gg in range(NUM_STATES // L):
                    s = tb + gg * L
                    m = jnp.minimum(
                        jnp.minimum(
                            buf[slot, 0, pl.ds(s, L)], buf[slot, 1, pl.ds(s, L)]
                        ),
                        jnp.minimum(
                            buf[slot, 2, pl.ds(s, L)], buf[slot, 3, pl.ds(s, L)]
                        ),
                    )
                    f = plsc.all_reduce_ffs(m > 0.0)
                    cands.append(jnp.where(f >= L, NUM_STATES, f + gg * L))
                best = jnp.minimum(
                    jnp.minimum(cands[0], cands[1]),
                    jnp.minimum(cands[2], cands[3]),
                )
                best = jnp.where(best == NUM_STATES, 0, best)
                acc = jnp.where(iota == k, best, acc)
            outbuf[pl.ds(blk * BT + g * L, L)] = acc.astype(jnp.float32)

    start(0, 0)
    start(1, 1)

    @pl.loop(0, NBLK, step=2)
    def _blk(blk):
        for b in range(2):
            drain(b)
            compute(blk + b, b)

            @pl.when(blk + b + 2 < NBLK)
            def _():
                s0 = (tok0 + (blk + b + 2) * BT) * NUM_STATES
                for t in range(NUM_TRIGGERS):
                    pltpu.async_copy(
                        scores_hbm.at[t, pl.ds(s0, BLK_ELEMS)],
                        buf.at[b, t],
                        sems[b],
                    )

    pltpu.sync_copy(outbuf, out_hbm.at[pl.ds(tok0, TOK_PER_W)])


@jax.jit
def _run(scores_flat):
    mesh = plsc.VectorSubcoreMesh(
        core_axis_name="c", subcore_axis_name="s", num_cores=NC, num_subcores=NS
    )
    return pl.kernel(
        _sc_body,
        out_type=jax.ShapeDtypeStruct((T,), jnp.float32),
        mesh=mesh,
        compiler_params=pltpu.CompilerParams(
            needs_layout_passes=False, disable_bounds_checks=True
        ),
        scratch_types=[
            pltpu.VMEM((2, NUM_TRIGGERS, BLK_ELEMS), jnp.float32),
            pltpu.VMEM((TOK_PER_W,), jnp.float32),
            pltpu.SemaphoreType.DMA,
            pltpu.SemaphoreType.DMA,
        ],
    )(scores_flat)


TC_BB = 2
TC_NB = BATCH // TC_BB


def _tc_body(x_ref, o_ref):
    x = x_ref[...]
    m = jnp.minimum(
        jnp.minimum(x[0], x[1]), jnp.minimum(x[2], x[3])
    )
    iota = lax.broadcasted_iota(
        jnp.int32, (TC_BB, NUM_STATES, SEQ), 1
    ).astype(jnp.float32)
    cand = jnp.where(m > 0.0, iota, jnp.float32(NUM_STATES))
    idx = jnp.min(cand, axis=1)
    idx = jnp.where(idx == jnp.float32(NUM_STATES), jnp.float32(0.0), idx)
    o_ref[...] = idx.reshape(TC_BB, 1, SEQ)


@jax.jit
def _run_tc(scores_t):
    return pl.pallas_call(
        _tc_body,
        grid=(TC_NB,),
        in_specs=[
            pl.BlockSpec(
                (NUM_TRIGGERS, TC_BB, NUM_STATES, SEQ), lambda i: (0, i, 0, 0)
            )
        ],
        out_specs=pl.BlockSpec((TC_BB, 1, SEQ), lambda i: (i, 0, 0)),
        out_shape=jax.ShapeDtypeStruct((BATCH, 1, SEQ), jnp.float32),
        compiler_params=pltpu.CompilerParams(
            dimension_semantics=("arbitrary",)
        ),
    )(scores_t)


def kernel(trigger_scores):
    scores_t = jnp.transpose(trigger_scores, (0, 1, 3, 2))
    return _run_tc(scores_t).reshape(BATCH, SEQ)


# speedup vs baseline: 5.5326x; 1.0033x over previous
import jax
import jax.numpy as jnp
from jax import lax
from jax.experimental import pallas as pl
from jax.experimental.pallas import tpu as pltpu

NUM_TRIGGERS = 4
BATCH = 32
SEQ = 2048
NUM_STATES = 64

TC_BB = 2
TC_NB = BATCH // TC_BB


def _tc_body(x_ref, o_ref):
    x = x_ref[...]
    m = jnp.minimum(
        jnp.minimum(x[0], x[1]), jnp.minimum(x[2], x[3])
    )
    iota = lax.broadcasted_iota(
        jnp.int32, (TC_BB, NUM_STATES, SEQ), 1
    ).astype(jnp.float32)
    cand = jnp.where(m > 0.0, iota, jnp.float32(NUM_STATES))
    idx = jnp.min(cand, axis=1)
    idx = jnp.where(idx == jnp.float32(NUM_STATES), jnp.float32(0.0), idx)
    o_ref[...] = idx.reshape(TC_BB, 1, SEQ)


@jax.jit
def _run_tc(scores_t):
    return pl.pallas_call(
        _tc_body,
        grid=(TC_NB,),
        in_specs=[
            pl.BlockSpec(
                (NUM_TRIGGERS, TC_BB, NUM_STATES, SEQ), lambda i: (0, i, 0, 0)
            )
        ],
        out_specs=pl.BlockSpec((TC_BB, 1, SEQ), lambda i: (i, 0, 0)),
        out_shape=jax.ShapeDtypeStruct((BATCH, 1, SEQ), jnp.float32),
        compiler_params=pltpu.CompilerParams(
            dimension_semantics=("arbitrary",)
        ),
    )(scores_t)


def kernel(trigger_scores):
    scores_t = jnp.transpose(trigger_scores, (0, 1, 3, 2))
    return _run_tc(scores_t).reshape(BATCH, SEQ)
